# baseline (speedup 1.0000x reference)
import functools

import jax
import jax.numpy as jnp
import numpy as np
from jax import lax
from jax.experimental import pallas as pl
from jax.experimental.pallas import tpu as pltpu
from jax.experimental.pallas import tpu_sc as plsc

N_FEATURES = 128
N_BASIS = 20
CUTOFF = 5.0
EPSILON = 1e-08


def _tc_body(d_ref, dv_ref, e20_ref, e3_ref, edge_ref, dvn_ref):
    d = d_ref[...]
    d20 = jnp.dot(d, e20_ref[...], preferred_element_type=jnp.float32)
    ji = lax.broadcasted_iota(jnp.int32, (1, d20.shape[1]), 1)
    freq = (jnp.pi / CUTOFF) * (lax.rem(ji, N_BASIS) + 1).astype(jnp.float32)
    edge_ref[...] = jnp.sin(d20 * freq) / (d20 + EPSILON)
    d3 = jnp.dot(d, e3_ref[...], preferred_element_type=jnp.float32)
    dvn_ref[...] = dv_ref[...] / (d3 + EPSILON)


@functools.partial(jax.jit, static_argnames=("rows", "nb"))
def _tc_radial(d2, dv2, rows, nb):
    R = 1024
    grid = rows // R
    e20 = jnp.asarray(np.repeat(np.eye(nb, dtype=np.float32), N_BASIS, axis=1))
    e3 = jnp.asarray(np.repeat(np.eye(nb, dtype=np.float32), 3, axis=1))
    return pl.pallas_call(
        _tc_body,
        grid=(grid,),
        in_specs=[
            pl.BlockSpec((R, nb), lambda i: (i, 0)),
            pl.BlockSpec((R, nb * 3), lambda i: (i, 0)),
            pl.BlockSpec((nb, nb * N_BASIS), lambda i: (0, 0)),
            pl.BlockSpec((nb, nb * 3), lambda i: (0, 0)),
        ],
        out_specs=[
            pl.BlockSpec((R, nb * N_BASIS), lambda i: (i, 0)),
            pl.BlockSpec((R, nb * 3), lambda i: (i, 0)),
        ],
        out_shape=[
            jax.ShapeDtypeStruct((rows, nb * N_BASIS), jnp.float32),
            jax.ShapeDtypeStruct((rows, nb * 3), jnp.float32),
        ],
    )(d2, dv2, e20, e3)


_NC, _NS = 2, 16
_NW = _NC * _NS


def _sc_gather_body(b_per_w, table_hbm, idx_hbm, out_hbm, idx_v, rows_v, sem):
    wid = lax.axis_index("s") * _NC + lax.axis_index("c")
    base = wid * b_per_w
    pltpu.sync_copy(idx_hbm.at[pl.ds(base, b_per_w)], idx_v)
    pltpu.async_copy(table_hbm.at[idx_v], rows_v, sem).wait()
    pltpu.sync_copy(rows_v, out_hbm.at[pl.ds(base, b_per_w)])


@functools.partial(jax.jit, static_argnames=("rows", "feat"))
def _sc_gather(table, idx, rows, feat):
    b_per_w = rows // _NW
    mesh = plsc.VectorSubcoreMesh(
        core_axis_name="c", subcore_axis_name="s", num_cores=_NC, num_subcores=_NS
    )
    return pl.kernel(
        functools.partial(_sc_gather_body, b_per_w),
        out_type=jax.ShapeDtypeStruct((rows, feat), jnp.float32),
        mesh=mesh,
        scratch_types=[
            pltpu.VMEM((b_per_w,), jnp.int32),
            pltpu.VMEM((b_per_w, feat), jnp.float32),
            pltpu.SemaphoreType.DMA,
        ],
    )(table, idx)


def kernel(atomic_numbers, positions, neighbor_mask, distances, distance_vectors,
           node_embedding_weight):
    B, A = atomic_numbers.shape
    NB = distances.shape[-1]
    F = node_embedding_weight.shape[-1]
    rows = B * A

    idx = atomic_numbers.reshape(rows).astype(jnp.int32)
    inv_node = _sc_gather(node_embedding_weight, idx, rows=rows, feat=F)

    d2 = distances.reshape(rows, NB)
    dv2 = distance_vectors.reshape(rows, NB * 3)
    edge2, dvn2 = _tc_radial(d2, dv2, rows=rows, nb=NB)

    invariant_node = inv_node.reshape(B, A, F)
    invariant_edge = edge2.reshape(B, A, NB, N_BASIS)
    dvn = dvn2.reshape(B, A, NB, 3)
    eq_F = jnp.zeros((B, A, 3), jnp.float32)
    eq_f = jnp.zeros((B, A, 3, F), jnp.float32)
    eq_dr = jnp.zeros((B, A, 3, F), jnp.float32)
    return (invariant_node, eq_F, eq_f, eq_dr, invariant_edge, distances, dvn)


# speedup vs baseline: 5.7988x; 5.7988x over previous
import functools
import math

import jax
import jax.numpy as jnp
from jax import lax
from jax.experimental import pallas as pl
from jax.experimental.pallas import tpu as pltpu
from jax.experimental.pallas import tpu_sc as plsc

N_FEATURES = 128
N_BASIS = 20
CUTOFF = 5.0
EPSILON = 1e-08


def _tc_body(d_ref, dv_ref, edge_ref, dvn_ref, dcp_ref):
    d = d_ref[0]
    theta = d * (math.pi / CUTOFF)
    s1 = jnp.sin(theta)
    c2 = 2.0 * jnp.cos(theta)
    rinv = 1.0 / (d + EPSILON)
    s_prev = jnp.zeros_like(d)
    s_cur = s1
    edge_ref[0, 0] = s1 * rinv
    for k in range(1, N_BASIS):
        s_next = c2 * s_cur - s_prev
        s_prev, s_cur = s_cur, s_next
        edge_ref[0, k] = s_cur * rinv
    for c in range(3):
        dvn_ref[0, c] = dv_ref[0, c] * rinv
    dcp_ref[0] = d


@functools.partial(jax.jit, static_argnames=("b", "nb", "a"))
def _tc_radial(d_t, dv_t, b, nb, a):
    return pl.pallas_call(
        _tc_body,
        grid=(b,),
        in_specs=[
            pl.BlockSpec((1, nb, a), lambda i: (i, 0, 0)),
            pl.BlockSpec((1, 3, nb, a), lambda i: (i, 0, 0, 0)),
        ],
        out_specs=[
            pl.BlockSpec((1, N_BASIS, nb, a), lambda i: (i, 0, 0, 0)),
            pl.BlockSpec((1, 3, nb, a), lambda i: (i, 0, 0, 0)),
            pl.BlockSpec((1, nb, a), lambda i: (i, 0, 0)),
        ],
        out_shape=[
            jax.ShapeDtypeStruct((b, N_BASIS, nb, a), jnp.float32),
            jax.ShapeDtypeStruct((b, 3, nb, a), jnp.float32),
            jax.ShapeDtypeStruct((b, nb, a), jnp.float32),
        ],
    )(d_t, dv_t)


_NC, _NS = 2, 16
_NW = _NC * _NS


def _sc_gather_body(b_per_w, table_hbm, idx_hbm, out_hbm, idx_v, rows_v, sem):
    wid = lax.axis_index("s") * _NC + lax.axis_index("c")
    base = wid * b_per_w
    pltpu.sync_copy(idx_hbm.at[pl.ds(base, b_per_w)], idx_v)
    pltpu.async_copy(table_hbm.at[idx_v], rows_v, sem).wait()
    pltpu.sync_copy(rows_v, out_hbm.at[pl.ds(base, b_per_w)])


@functools.partial(jax.jit, static_argnames=("rows", "feat"))
def _sc_gather(table, idx, rows, feat):
    b_per_w = rows // _NW
    mesh = plsc.VectorSubcoreMesh(
        core_axis_name="c", subcore_axis_name="s", num_cores=_NC, num_subcores=_NS
    )
    return pl.kernel(
        functools.partial(_sc_gather_body, b_per_w),
        out_type=jax.ShapeDtypeStruct((rows, feat), jnp.float32),
        mesh=mesh,
        scratch_types=[
            pltpu.VMEM((b_per_w,), jnp.int32),
            pltpu.VMEM((b_per_w, feat), jnp.float32),
            pltpu.SemaphoreType.DMA,
        ],
    )(table, idx)


def kernel(atomic_numbers, positions, neighbor_mask, distances, distance_vectors,
           node_embedding_weight):
    B, A = atomic_numbers.shape
    NB = distances.shape[-1]
    F = node_embedding_weight.shape[-1]
    rows = B * A

    idx = atomic_numbers.reshape(rows).astype(jnp.int32)
    inv_node = _sc_gather(node_embedding_weight, idx, rows=rows, feat=F)

    d_t = jnp.transpose(distances, (0, 2, 1))
    dv_t = jnp.transpose(distance_vectors, (0, 3, 2, 1))
    edge_t, dvn_t, dcp_t = _tc_radial(d_t, dv_t, b=B, nb=NB, a=A)

    invariant_node = inv_node.reshape(B, A, F)
    invariant_edge = jnp.transpose(edge_t, (0, 3, 2, 1))
    dvn = jnp.transpose(dvn_t, (0, 3, 2, 1))
    d_out = jnp.transpose(dcp_t, (0, 2, 1))
    eq_F = jnp.zeros((B, A, 3), jnp.float32)
    eq_f = jnp.zeros((B, A, 3, F), jnp.float32)
    eq_dr = jnp.zeros((B, A, 3, F), jnp.float32)
    return (invariant_node, eq_F, eq_f, eq_dr, invariant_edge, d_out, dvn)


# speedup vs baseline: 6.2730x; 1.0818x over previous
import functools
import math

import jax
import jax.numpy as jnp
from jax import lax
from jax.experimental import pallas as pl
from jax.experimental.pallas import tpu as pltpu
from jax.experimental.pallas import tpu_sc as plsc

N_FEATURES = 128
N_BASIS = 20
CUTOFF = 5.0
EPSILON = 1e-08


def _tc_body(d_ref, dv_ref, edge_ref, dvn_ref, dcp_ref, zf_ref, zdr_ref):
    d = d_ref[0]
    theta = d * (math.pi / CUTOFF)
    s1 = jnp.sin(theta)
    c2 = 2.0 * jnp.cos(theta)
    rinv = 1.0 / (d + EPSILON)
    s_prev = jnp.zeros_like(d)
    s_cur = s1
    edge_ref[0, 0] = s1 * rinv
    for k in range(1, N_BASIS):
        s_next = c2 * s_cur - s_prev
        s_prev, s_cur = s_cur, s_next
        edge_ref[0, k] = s_cur * rinv
    for c in range(3):
        dvn_ref[0, c] = dv_ref[0, c] * rinv
    dcp_ref[0] = d
    zf_ref[...] = jnp.zeros_like(zf_ref)
    zdr_ref[...] = jnp.zeros_like(zdr_ref)


@functools.partial(jax.jit, static_argnames=("b", "nb", "a", "f"))
def _tc_radial(d_t, dv_t, b, nb, a, f):
    return pl.pallas_call(
        _tc_body,
        grid=(b,),
        in_specs=[
            pl.BlockSpec((1, nb, a), lambda i: (i, 0, 0)),
            pl.BlockSpec((1, 3, nb, a), lambda i: (i, 0, 0, 0)),
        ],
        out_specs=[
            pl.BlockSpec((1, N_BASIS, nb, a), lambda i: (i, 0, 0, 0)),
            pl.BlockSpec((1, 3, nb, a), lambda i: (i, 0, 0, 0)),
            pl.BlockSpec((1, nb, a), lambda i: (i, 0, 0)),
            pl.BlockSpec((1, 3, a, f), lambda i: (i, 0, 0, 0)),
            pl.BlockSpec((1, 3, a, f), lambda i: (i, 0, 0, 0)),
        ],
        out_shape=[
            jax.ShapeDtypeStruct((b, N_BASIS, nb, a), jnp.float32),
            jax.ShapeDtypeStruct((b, 3, nb, a), jnp.float32),
            jax.ShapeDtypeStruct((b, nb, a), jnp.float32),
            jax.ShapeDtypeStruct((b, 3, a, f), jnp.float32),
            jax.ShapeDtypeStruct((b, 3, a, f), jnp.float32),
        ],
    )(d_t, dv_t)


_NC, _NS = 2, 16
_NW = _NC * _NS


def _sc_gather_body(b_per_w, table_hbm, idx_hbm, out_hbm, idx_v, rows_v, sem):
    wid = lax.axis_index("s") * _NC + lax.axis_index("c")
    base = wid * b_per_w
    pltpu.sync_copy(idx_hbm.at[pl.ds(base, b_per_w)], idx_v)
    pltpu.async_copy(table_hbm.at[idx_v], rows_v, sem).wait()
    pltpu.sync_copy(rows_v, out_hbm.at[pl.ds(base, b_per_w)])


@functools.partial(jax.jit, static_argnames=("rows", "feat"))
def _sc_gather(table, idx, rows, feat):
    b_per_w = rows // _NW
    mesh = plsc.VectorSubcoreMesh(
        core_axis_name="c", subcore_axis_name="s", num_cores=_NC, num_subcores=_NS
    )
    return pl.kernel(
        functools.partial(_sc_gather_body, b_per_w),
        out_type=jax.ShapeDtypeStruct((rows, feat), jnp.float32),
        mesh=mesh,
        scratch_types=[
            pltpu.VMEM((b_per_w,), jnp.int32),
            pltpu.VMEM((b_per_w, feat), jnp.float32),
            pltpu.SemaphoreType.DMA,
        ],
    )(table, idx)


def kernel(atomic_numbers, positions, neighbor_mask, distances, distance_vectors,
           node_embedding_weight):
    B, A = atomic_numbers.shape
    NB = distances.shape[-1]
    F = node_embedding_weight.shape[-1]
    rows = B * A

    idx = atomic_numbers.reshape(rows).astype(jnp.int32)
    inv_node = _sc_gather(node_embedding_weight, idx, rows=rows, feat=F)

    d_t = jnp.transpose(distances, (0, 2, 1))
    dv_t = jnp.transpose(distance_vectors, (0, 3, 2, 1))
    edge_t, dvn_t, dcp_t, zf_t, zdr_t = _tc_radial(d_t, dv_t, b=B, nb=NB, a=A, f=F)

    invariant_node = inv_node.reshape(B, A, F)
    invariant_edge = jnp.transpose(edge_t, (0, 3, 2, 1))
    dvn = jnp.transpose(dvn_t, (0, 3, 2, 1))
    d_out = jnp.transpose(dcp_t, (0, 2, 1))
    eq_F = jnp.zeros((B, A, 3), jnp.float32)
    eq_f = jnp.transpose(zf_t, (0, 2, 1, 3))
    eq_dr = jnp.transpose(zdr_t, (0, 2, 1, 3))
    return (invariant_node, eq_F, eq_f, eq_dr, invariant_edge, d_out, dvn)
